# baseline (speedup 1.0000x reference)
import functools

import jax
import jax.numpy as jnp
from jax import lax
from jax.experimental import pallas as pl
from jax.experimental.pallas import tpu as pltpu
from jax.experimental.pallas import tpu_sc as plsc

H = 128
NB = 4
DEPTH = 6

_NC, _NS, _L = 2, 16, 16
_NW = _NC * _NS


def _sc_invar_body(widx_hbm, tabe_hbm, tabzh_hbm, exr_hbm, xzh_hbm,
                   widx0, widx1, erows0, erows1, zrows0, zrows1,
                   semA0, semA1, semS0, semS1, *, ew, be):
    wid = lax.axis_index("s") * _NC + lax.axis_index("c")
    w0 = wid * ew
    nblk = ew // be
    tail = ew - nblk * be

    def fire(widx_v, erows_v, zrows_v, semA, base, cnt):
        pltpu.sync_copy(widx_hbm.at[pl.ds(base, cnt)],
                        widx_v.at[pl.ds(0, cnt)])
        pltpu.async_copy(tabe_hbm.at[widx_v.at[pl.ds(0, cnt)]],
                         erows_v.at[pl.ds(0, cnt)], semA)
        pltpu.async_copy(tabzh_hbm.at[widx_v.at[pl.ds(0, cnt)]],
                         zrows_v.at[pl.ds(0, cnt)], semA)

    def store(widx_v, erows_v, zrows_v, semA, semS, base, cnt):
        pltpu.make_async_copy(tabe_hbm.at[widx_v.at[pl.ds(0, cnt)]],
                              erows_v.at[pl.ds(0, cnt)], semA).wait()
        pltpu.make_async_copy(tabzh_hbm.at[widx_v.at[pl.ds(0, cnt)]],
                              zrows_v.at[pl.ds(0, cnt)], semA).wait()
        pltpu.async_copy(erows_v.at[pl.ds(0, cnt)],
                         exr_hbm.at[pl.ds(base, cnt)], semS)
        pltpu.async_copy(zrows_v.at[pl.ds(0, cnt)],
                         xzh_hbm.at[pl.ds(base, cnt)], semS)

    def wait_store(erows_v, zrows_v, semS, cnt):
        pltpu.make_async_copy(erows_v.at[pl.ds(0, cnt)],
                              exr_hbm.at[pl.ds(0, cnt)], semS).wait()
        pltpu.make_async_copy(zrows_v.at[pl.ds(0, cnt)],
                              xzh_hbm.at[pl.ds(0, cnt)], semS).wait()

    fire(widx0, erows0, zrows0, semA0, w0, be)
    fire(widx1, erows1, zrows1, semA1, w0 + be, be)
    store(widx0, erows0, zrows0, semA0, semS0, w0, be)

    def pair(i, c):
        b0 = w0 + (2 * i) * be
        b1 = b0 + be
        wait_store(erows0, zrows0, semS0, be)
        fire(widx0, erows0, zrows0, semA0, b0, be)
        store(widx1, erows1, zrows1, semA1, semS1, b1 - 2 * be, be)
        wait_store(erows1, zrows1, semS1, be)
        fire(widx1, erows1, zrows1, semA1, b1, be)
        store(widx0, erows0, zrows0, semA0, semS0, b0, be)
        return c

    lax.fori_loop(1, nblk // 2, pair, 0)
    store(widx1, erows1, zrows1, semA1, semS1, w0 + (nblk - 2) * be, be)
    wait_store(erows0, zrows0, semS0, be)
    fire(widx0, erows0, zrows0, semA0, w0 + (nblk - 1) * be, be)
    store(widx0, erows0, zrows0, semA0, semS0, w0 + (nblk - 1) * be, be)
    wait_store(erows1, zrows1, semS1, be)
    if tail:
        fire(widx1, erows1, zrows1, semA1, w0 + nblk * be, tail)
        store(widx1, erows1, zrows1, semA1, semS1, w0 + nblk * be, tail)
        wait_store(erows1, zrows1, semS1, tail)
    wait_store(erows0, zrows0, semS0, be)


def _sc_invariants(widx, tabe, tabzh, *, be=128):
    E = widx.shape[0]
    ew = E // _NW
    mesh = plsc.VectorSubcoreMesh(core_axis_name="c", subcore_axis_name="s")
    f = pl.kernel(
        functools.partial(_sc_invar_body, ew=ew, be=be),
        out_type=(jax.ShapeDtypeStruct((E, H), jnp.float32),
                  jax.ShapeDtypeStruct((E, 2 * H), jnp.float32)),
        mesh=mesh,
        scratch_types=[
            pltpu.VMEM((be,), jnp.int32),
            pltpu.VMEM((be,), jnp.int32),
            pltpu.VMEM((be, H), jnp.float32),
            pltpu.VMEM((be, H), jnp.float32),
            pltpu.VMEM((be, 2 * H), jnp.float32),
            pltpu.VMEM((be, 2 * H), jnp.float32),
            pltpu.SemaphoreType.DMA,
            pltpu.SemaphoreType.DMA,
            pltpu.SemaphoreType.DMA,
            pltpu.SemaphoreType.DMA,
        ],
    )
    return f(widx, tabe, tabzh)


def _sc_root_body(snmgf_hbm, swid_hbm, msgs_hbm, tabo_hbm, msum_hbm, eo_hbm,
                  idx_v, rows_v, widx_v, orows_v, msum_v, sem, *, bw):
    wid = lax.axis_index("s") * _NC + lax.axis_index("c")
    base = wid * bw
    pltpu.sync_copy(snmgf_hbm.at[pl.ds(base * NB, bw * NB)], idx_v)
    cp = pltpu.async_copy(msgs_hbm.at[idx_v], rows_v, sem)
    pltpu.sync_copy(swid_hbm.at[pl.ds(base, bw)], widx_v)
    cp.wait()
    pltpu.async_copy(tabo_hbm.at[widx_v], orows_v, sem).wait()
    for nloc in range(bw):
        r0 = nloc * NB
        for j in range(H // _L):
            o = j * _L
            s = rows_v[r0, pl.ds(o, _L)]
            for m in range(1, NB):
                s = s + rows_v[r0 + m, pl.ds(o, _L)]
            msum_v[nloc, pl.ds(o, _L)] = s
    pltpu.sync_copy(msum_v, msum_hbm.at[pl.ds(base, bw)])
    pltpu.sync_copy(orows_v, eo_hbm.at[pl.ds(base, bw)])


def _sc_root(snmgf, swid, msgs, tabo):
    B = swid.shape[0]
    bw = B // _NW
    mesh = plsc.VectorSubcoreMesh(core_axis_name="c", subcore_axis_name="s")
    f = pl.kernel(
        functools.partial(_sc_root_body, bw=bw),
        out_type=(jax.ShapeDtypeStruct((B, H), jnp.float32),
                  jax.ShapeDtypeStruct((B, H), jnp.float32)),
        mesh=mesh,
        scratch_types=[
            pltpu.VMEM((bw * NB,), jnp.int32),
            pltpu.VMEM((bw * NB, H), jnp.float32),
            pltpu.VMEM((bw,), jnp.int32),
            pltpu.VMEM((bw, H), jnp.float32),
            pltpu.VMEM((bw, H), jnp.float32),
            pltpu.SemaphoreType.DMA,
        ],
    )
    return f(snmgf, swid, msgs, tabo)


def _sc_nei_body(heu_hbm, exr_hbm, adjf_hbm, sums_hbm,
                 idx0, idx1, rows0, rows1, exr0, exr1, out0, out1,
                 semA0, semA1, semS0, semS1, *, ew, be):
    wid = lax.axis_index("s") * _NC + lax.axis_index("c")
    w0 = wid * ew
    nblk = ew // be

    def fire(idx_v, rows_v, exr_v, semA, base):
        pltpu.sync_copy(adjf_hbm.at[pl.ds(base * NB, be * NB)], idx_v)
        pltpu.async_copy(heu_hbm.at[idx_v], rows_v, semA)
        pltpu.async_copy(exr_hbm.at[pl.ds(base, be)], exr_v, semA)

    def wait_store(out_v, semS):
        pltpu.make_async_copy(out_v, sums_hbm.at[pl.ds(0, be)], semS).wait()

    def compute(idx_v, rows_v, exr_v, out_v, semA, semS, base):
        pltpu.make_async_copy(heu_hbm.at[idx_v], rows_v, semA).wait()
        pltpu.make_async_copy(exr_hbm.at[pl.ds(base, be)], exr_v, semA).wait()

        @plsc.parallel_loop(0, be, 1, unroll=2)
        def _edge(e):
            r0 = e * NB
            for j in range(H // _L):
                o = j * _L
                ex16 = exr_v[e, pl.ds(o, _L)]
                s = jnp.zeros((_L,), jnp.float32)
                g = jnp.zeros((_L,), jnp.float32)
                for m in range(NB):
                    hv = rows_v[r0 + m, pl.ds(o, _L)]
                    eu = rows_v[r0 + m, pl.ds(H + o, _L)]
                    s = s + hv
                    g = g + hv / (1.0 + ex16 * eu)
                out_v[e, pl.ds(o, _L)] = s
                out_v[e, pl.ds(H + o, _L)] = g

        pltpu.async_copy(out_v, sums_hbm.at[pl.ds(base, be)], semS)

    fire(idx0, rows0, exr0, semA0, w0)
    fire(idx1, rows1, exr1, semA1, w0 + be)
    compute(idx0, rows0, exr0, out0, semA0, semS0, w0)
    fire(idx0, rows0, exr0, semA0, w0 + 2 * be)
    compute(idx1, rows1, exr1, out1, semA1, semS1, w0 + be)

    def pair(i, carry):
        b0 = w0 + (2 * i) * be
        b1 = b0 + be
        fire(idx1, rows1, exr1, semA1, b1)
        wait_store(out0, semS0)
        compute(idx0, rows0, exr0, out0, semA0, semS0, b0)
        fire(idx0, rows0, exr0, semA0, b1 + be)
        wait_store(out1, semS1)
        compute(idx1, rows1, exr1, out1, semA1, semS1, b1)
        return carry

    lax.fori_loop(1, nblk // 2, pair, 0)
    wait_store(out0, semS0)
    compute(idx0, rows0, exr0, out0, semA0, semS0, w0 + (nblk - 1) * be)
    wait_store(out0, semS0)
    wait_store(out1, semS1)


def _sc_neighbor(heu, exr, adjf, *, be=40):
    E = exr.shape[0]
    ew = E // _NW
    mesh = plsc.VectorSubcoreMesh(core_axis_name="c", subcore_axis_name="s")
    f = pl.kernel(
        functools.partial(_sc_nei_body, ew=ew, be=be),
        out_type=jax.ShapeDtypeStruct((E, 2 * H), jnp.float32),
        mesh=mesh,
        scratch_types=[
            pltpu.VMEM((be * NB,), jnp.int32),
            pltpu.VMEM((be * NB,), jnp.int32),
            pltpu.VMEM((be * NB, 2 * H), jnp.float32),
            pltpu.VMEM((be * NB, 2 * H), jnp.float32),
            pltpu.VMEM((be, H), jnp.float32),
            pltpu.VMEM((be, H), jnp.float32),
            pltpu.VMEM((be, 2 * H), jnp.float32),
            pltpu.VMEM((be, 2 * H), jnp.float32),
            pltpu.SemaphoreType.DMA,
            pltpu.SemaphoreType.DMA,
            pltpu.SemaphoreType.DMA,
            pltpu.SemaphoreType.DMA,
        ],
    )
    return f(heu, exr, adjf)


def _tables_body(emb_ref, wr_ref, wzx_ref, whx_ref, wox_ref, bz_ref, bh_ref,
                 bo_ref, tabe_ref, tabzh_ref, tabo_ref):
    emb = emb_ref[...]
    tabe_ref[...] = jnp.exp(-(emb @ wr_ref[...]))
    tabzh_ref[:, :H] = emb @ wzx_ref[...] + bz_ref[...]
    tabzh_ref[:, H:] = emb @ whx_ref[...] + bh_ref[...]
    tabo_ref[...] = emb @ wox_ref[...] + bo_ref[...]


def _make_tables(embedding, W_r, Wz_x, Wh_x, Wo_x, b_z, b_h, b_out):
    V = embedding.shape[0]
    return pl.pallas_call(
        _tables_body,
        out_shape=(jax.ShapeDtypeStruct((V, H), jnp.float32),
                   jax.ShapeDtypeStruct((V, 2 * H), jnp.float32),
                   jax.ShapeDtypeStruct((V, H), jnp.float32)),
    )(embedding, W_r.T, Wz_x.T, Wh_x.T, Wo_x.T,
      b_z[None, :], b_h[None, :], b_out[None, :])


def _gru_h(xzh_ref, sums_ref, wzh_ref, whh_ref, be):
    sumh = sums_ref[:, :H]
    z = jax.nn.sigmoid(xzh_ref[:, :H] + sumh @ wzh_ref[...])
    p = jnp.tanh(xzh_ref[:, H:] + sums_ref[:, H:] @ whh_ref[...])
    h = (1.0 - z) * sumh + z * p
    row = pl.program_id(0) * be + lax.broadcasted_iota(jnp.int32, (be, 1), 0)
    return jnp.where(row > 0, h, 0.0)


def _gru_body(xzh_ref, sums_ref, wzh_ref, whh_ref, ur_ref,
              bur_ref, hhu_ref, *, be):
    h = _gru_h(xzh_ref, sums_ref, wzh_ref, whh_ref, be)
    hhu_ref[:, :H] = h
    hhu_ref[:, H:] = jnp.exp(-(h @ ur_ref[...] + bur_ref[...]))


def _gru_last_body(xzh_ref, sums_ref, wzh_ref, whh_ref, h_ref, *, be):
    h_ref[...] = _gru_h(xzh_ref, sums_ref, wzh_ref, whh_ref, be)


def _gru_dense(xzh, sums, wzh, whh, ur_t, bur, *, be=2000):
    E = sums.shape[0]
    bs_e2 = pl.BlockSpec((be, 2 * H), lambda i: (i, 0))
    bs_w = pl.BlockSpec((H, H), lambda i: (0, 0))
    return pl.pallas_call(
        functools.partial(_gru_body, be=be),
        grid=(E // be,),
        in_specs=[bs_e2, bs_e2, bs_w, bs_w, bs_w,
                  pl.BlockSpec((1, H), lambda i: (0, 0))],
        out_specs=bs_e2,
        out_shape=jax.ShapeDtypeStruct((E, 2 * H), jnp.float32),
    )(xzh, sums, wzh, whh, ur_t, bur[None, :])


def _gru_dense_last(xzh, sums, wzh, whh, *, be=2000):
    E = sums.shape[0]
    bs_e2 = pl.BlockSpec((be, 2 * H), lambda i: (i, 0))
    bs_w = pl.BlockSpec((H, H), lambda i: (0, 0))
    return pl.pallas_call(
        functools.partial(_gru_last_body, be=be),
        grid=(E // be,),
        in_specs=[bs_e2, bs_e2, bs_w, bs_w],
        out_specs=pl.BlockSpec((be, H), lambda i: (i, 0)),
        out_shape=jax.ShapeDtypeStruct((E, H), jnp.float32),
    )(xzh, sums, wzh, whh)


def _root_body(eo_ref, msum_ref, woh_ref, tv_ref):
    tv_ref[...] = jnp.maximum(eo_ref[...] + msum_ref[...] @ woh_ref[...], 0.0)


def _root_out(eo, msum, woh_t):
    B = msum.shape[0]
    return pl.pallas_call(
        _root_body,
        out_shape=jax.ShapeDtypeStruct((B, H), jnp.float32),
    )(eo, msum, woh_t)


def kernel(node_wid_list, edge_node_idx_list, node_message_graph,
           mess_adjacency_graph, scope, embedding, W_z, b_z, W_r, U_r, b_Ur,
           W_h, b_h, W_out, b_out):
    E = mess_adjacency_graph.shape[0]
    N = node_wid_list.shape[0]

    Wz_x, Wz_h = W_z[:, :H], W_z[:, H:]
    Wh_x, Wh_h = W_h[:, :H], W_h[:, H:]
    Wo_x, Wo_h = W_out[:, :H], W_out[:, H:]

    tabe, tabzh, tabo = _make_tables(embedding, W_r, Wz_x, Wh_x, Wo_x,
                                     b_z, b_h, b_out)

    widx = jnp.take(node_wid_list, edge_node_idx_list, axis=0)
    exr, xzh = _sc_invariants(widx, tabe, tabzh)

    hhu = jnp.concatenate(
        [jnp.zeros((E, H), jnp.float32),
         jnp.broadcast_to(jnp.exp(-b_Ur)[None, :], (E, H))], axis=1)

    wzh_t, whh_t, ur_t, woh_t = Wz_h.T, Wh_h.T, U_r.T, Wo_h.T
    adjf = mess_adjacency_graph.reshape(E * NB)

    for it in range(DEPTH):
        sums = _sc_neighbor(hhu, exr, adjf)
        if it < DEPTH - 1:
            hhu = _gru_dense(xzh, sums, wzh_t, whh_t, ur_t, b_Ur)
        else:
            messages = _gru_dense_last(xzh, sums, wzh_t, whh_t)

    scope0 = scope[:, 0]
    swid = jnp.take(node_wid_list, scope0, axis=0)
    snmgf = jnp.take(node_message_graph, scope0, axis=0).reshape(-1)
    msum, eo = _sc_root(snmgf, swid, messages, tabo)
    tree_vecs = _root_out(eo, msum, woh_t)
    return (tree_vecs, messages)


# speedup vs baseline: 1.0149x; 1.0149x over previous
import functools

import jax
import jax.numpy as jnp
from jax import lax
from jax.experimental import pallas as pl
from jax.experimental.pallas import tpu as pltpu
from jax.experimental.pallas import tpu_sc as plsc

H = 128
NB = 4
DEPTH = 6

_NC, _NS, _L = 2, 16, 16
_NW = _NC * _NS


def _sc_invar_body(widx_hbm, tabe_hbm, tabzh_hbm, exr_hbm, xzh_hbm,
                   widx0, widx1, erows0, erows1, zrows0, zrows1,
                   semA0, semA1, semS0, semS1, *, ew, be):
    wid = lax.axis_index("s") * _NC + lax.axis_index("c")
    w0 = wid * ew
    nblk = ew // be
    tail = ew - nblk * be

    def fire(widx_v, erows_v, zrows_v, semA, base, cnt):
        pltpu.sync_copy(widx_hbm.at[pl.ds(base, cnt)],
                        widx_v.at[pl.ds(0, cnt)])
        pltpu.async_copy(tabe_hbm.at[widx_v.at[pl.ds(0, cnt)]],
                         erows_v.at[pl.ds(0, cnt)], semA)
        pltpu.async_copy(tabzh_hbm.at[widx_v.at[pl.ds(0, cnt)]],
                         zrows_v.at[pl.ds(0, cnt)], semA)

    def store(widx_v, erows_v, zrows_v, semA, semS, base, cnt):
        pltpu.make_async_copy(tabe_hbm.at[widx_v.at[pl.ds(0, cnt)]],
                              erows_v.at[pl.ds(0, cnt)], semA).wait()
        pltpu.make_async_copy(tabzh_hbm.at[widx_v.at[pl.ds(0, cnt)]],
                              zrows_v.at[pl.ds(0, cnt)], semA).wait()
        pltpu.async_copy(erows_v.at[pl.ds(0, cnt)],
                         exr_hbm.at[pl.ds(base, cnt)], semS)
        pltpu.async_copy(zrows_v.at[pl.ds(0, cnt)],
                         xzh_hbm.at[pl.ds(base, cnt)], semS)

    def wait_store(erows_v, zrows_v, semS, cnt):
        pltpu.make_async_copy(erows_v.at[pl.ds(0, cnt)],
                              exr_hbm.at[pl.ds(0, cnt)], semS).wait()
        pltpu.make_async_copy(zrows_v.at[pl.ds(0, cnt)],
                              xzh_hbm.at[pl.ds(0, cnt)], semS).wait()

    fire(widx0, erows0, zrows0, semA0, w0, be)
    fire(widx1, erows1, zrows1, semA1, w0 + be, be)
    store(widx0, erows0, zrows0, semA0, semS0, w0, be)

    def pair(i, c):
        b0 = w0 + (2 * i) * be
        b1 = b0 + be
        wait_store(erows0, zrows0, semS0, be)
        fire(widx0, erows0, zrows0, semA0, b0, be)
        store(widx1, erows1, zrows1, semA1, semS1, b1 - 2 * be, be)
        wait_store(erows1, zrows1, semS1, be)
        fire(widx1, erows1, zrows1, semA1, b1, be)
        store(widx0, erows0, zrows0, semA0, semS0, b0, be)
        return c

    lax.fori_loop(1, nblk // 2, pair, 0)
    store(widx1, erows1, zrows1, semA1, semS1, w0 + (nblk - 2) * be, be)
    wait_store(erows0, zrows0, semS0, be)
    fire(widx0, erows0, zrows0, semA0, w0 + (nblk - 1) * be, be)
    store(widx0, erows0, zrows0, semA0, semS0, w0 + (nblk - 1) * be, be)
    wait_store(erows1, zrows1, semS1, be)
    if tail:
        fire(widx1, erows1, zrows1, semA1, w0 + nblk * be, tail)
        store(widx1, erows1, zrows1, semA1, semS1, w0 + nblk * be, tail)
        wait_store(erows1, zrows1, semS1, tail)
    wait_store(erows0, zrows0, semS0, be)


def _sc_invariants(widx, tabe, tabzh, *, be=128):
    E = widx.shape[0]
    ew = E // _NW
    mesh = plsc.VectorSubcoreMesh(core_axis_name="c", subcore_axis_name="s")
    f = pl.kernel(
        functools.partial(_sc_invar_body, ew=ew, be=be),
        out_type=(jax.ShapeDtypeStruct((E, H), jnp.float32),
                  jax.ShapeDtypeStruct((E, 2 * H), jnp.float32)),
        mesh=mesh,
        scratch_types=[
            pltpu.VMEM((be,), jnp.int32),
            pltpu.VMEM((be,), jnp.int32),
            pltpu.VMEM((be, H), jnp.float32),
            pltpu.VMEM((be, H), jnp.float32),
            pltpu.VMEM((be, 2 * H), jnp.float32),
            pltpu.VMEM((be, 2 * H), jnp.float32),
            pltpu.SemaphoreType.DMA,
            pltpu.SemaphoreType.DMA,
            pltpu.SemaphoreType.DMA,
            pltpu.SemaphoreType.DMA,
        ],
    )
    return f(widx, tabe, tabzh)


def _sc_root_body(snmgf_hbm, swid_hbm, msgs_hbm, tabo_hbm, msum_hbm, eo_hbm,
                  idx_v, rows_v, widx_v, orows_v, msum_v, sem, *, bw):
    wid = lax.axis_index("s") * _NC + lax.axis_index("c")
    base = wid * bw
    pltpu.sync_copy(snmgf_hbm.at[pl.ds(base * NB, bw * NB)], idx_v)
    cp = pltpu.async_copy(msgs_hbm.at[idx_v], rows_v, sem)
    pltpu.sync_copy(swid_hbm.at[pl.ds(base, bw)], widx_v)
    cp.wait()
    pltpu.async_copy(tabo_hbm.at[widx_v], orows_v, sem).wait()
    for nloc in range(bw):
        r0 = nloc * NB
        for j in range(H // _L):
            o = j * _L
            s = rows_v[r0, pl.ds(o, _L)]
            for m in range(1, NB):
                s = s + rows_v[r0 + m, pl.ds(o, _L)]
            msum_v[nloc, pl.ds(o, _L)] = s
    pltpu.sync_copy(msum_v, msum_hbm.at[pl.ds(base, bw)])
    pltpu.sync_copy(orows_v, eo_hbm.at[pl.ds(base, bw)])


def _sc_root(snmgf, swid, msgs, tabo):
    B = swid.shape[0]
    bw = B // _NW
    mesh = plsc.VectorSubcoreMesh(core_axis_name="c", subcore_axis_name="s")
    f = pl.kernel(
        functools.partial(_sc_root_body, bw=bw),
        out_type=(jax.ShapeDtypeStruct((B, H), jnp.float32),
                  jax.ShapeDtypeStruct((B, H), jnp.float32)),
        mesh=mesh,
        scratch_types=[
            pltpu.VMEM((bw * NB,), jnp.int32),
            pltpu.VMEM((bw * NB, H), jnp.float32),
            pltpu.VMEM((bw,), jnp.int32),
            pltpu.VMEM((bw, H), jnp.float32),
            pltpu.VMEM((bw, H), jnp.float32),
            pltpu.SemaphoreType.DMA,
        ],
    )
    return f(snmgf, swid, msgs, tabo)


def _sc_nei_body(heu_hbm, exr_hbm, adjf_hbm, sums_hbm,
                 idx0, idx1, rows0, rows1, exr0, exr1, out0, out1,
                 semA0, semA1, semS0, semS1, *, ew, be):
    wid = lax.axis_index("s") * _NC + lax.axis_index("c")
    w0 = wid * ew
    nblk = ew // be

    def fire(idx_v, rows_v, exr_v, semA, base):
        pltpu.sync_copy(adjf_hbm.at[pl.ds(base * NB, be * NB)], idx_v)
        pltpu.async_copy(heu_hbm.at[idx_v], rows_v, semA)
        pltpu.async_copy(exr_hbm.at[pl.ds(base, be)], exr_v, semA)

    def wait_store(out_v, semS):
        pltpu.make_async_copy(out_v, sums_hbm.at[pl.ds(0, be)], semS).wait()

    def compute(idx_v, rows_v, exr_v, out_v, semA, semS, base):
        pltpu.make_async_copy(heu_hbm.at[idx_v], rows_v, semA).wait()
        pltpu.make_async_copy(exr_hbm.at[pl.ds(base, be)], exr_v, semA).wait()

        @plsc.parallel_loop(0, be, 1, unroll=2)
        def _edge(e):
            r0 = e * NB
            for j in range(H // _L):
                o = j * _L
                ex16 = exr_v[e, pl.ds(o, _L)]
                s = jnp.zeros((_L,), jnp.float32)
                g = jnp.zeros((_L,), jnp.float32)
                for m in range(NB):
                    hv = rows_v[r0 + m, pl.ds(o, _L)]
                    eu = rows_v[r0 + m, pl.ds(H + o, _L)]
                    s = s + hv
                    g = g + hv / (1.0 + ex16 * eu)
                out_v[e, pl.ds(o, _L)] = s
                out_v[e, pl.ds(H + o, _L)] = g

        pltpu.async_copy(out_v, sums_hbm.at[pl.ds(base, be)], semS)

    fire(idx0, rows0, exr0, semA0, w0)
    fire(idx1, rows1, exr1, semA1, w0 + be)
    compute(idx0, rows0, exr0, out0, semA0, semS0, w0)
    fire(idx0, rows0, exr0, semA0, w0 + 2 * be)
    compute(idx1, rows1, exr1, out1, semA1, semS1, w0 + be)

    def pair(i, carry):
        b0 = w0 + (2 * i) * be
        b1 = b0 + be
        fire(idx1, rows1, exr1, semA1, b1)
        wait_store(out0, semS0)
        compute(idx0, rows0, exr0, out0, semA0, semS0, b0)
        fire(idx0, rows0, exr0, semA0, b1 + be)
        wait_store(out1, semS1)
        compute(idx1, rows1, exr1, out1, semA1, semS1, b1)
        return carry

    lax.fori_loop(1, nblk // 2, pair, 0)
    wait_store(out0, semS0)
    compute(idx0, rows0, exr0, out0, semA0, semS0, w0 + (nblk - 1) * be)
    wait_store(out0, semS0)
    wait_store(out1, semS1)


def _sc_neighbor(heu, exr, adjf, *, be=40):
    E = exr.shape[0]
    ew = E // _NW
    mesh = plsc.VectorSubcoreMesh(core_axis_name="c", subcore_axis_name="s")
    f = pl.kernel(
        functools.partial(_sc_nei_body, ew=ew, be=be),
        out_type=jax.ShapeDtypeStruct((E, 2 * H), jnp.float32),
        mesh=mesh,
        scratch_types=[
            pltpu.VMEM((be * NB,), jnp.int32),
            pltpu.VMEM((be * NB,), jnp.int32),
            pltpu.VMEM((be * NB, 2 * H), jnp.float32),
            pltpu.VMEM((be * NB, 2 * H), jnp.float32),
            pltpu.VMEM((be, H), jnp.float32),
            pltpu.VMEM((be, H), jnp.float32),
            pltpu.VMEM((be, 2 * H), jnp.float32),
            pltpu.VMEM((be, 2 * H), jnp.float32),
            pltpu.SemaphoreType.DMA,
            pltpu.SemaphoreType.DMA,
            pltpu.SemaphoreType.DMA,
            pltpu.SemaphoreType.DMA,
        ],
    )
    return f(heu, exr, adjf)


def _tables_body(emb_ref, wr_ref, wzx_ref, whx_ref, wox_ref, bz_ref, bh_ref,
                 bo_ref, tabe_ref, tabzh_ref, tabo_ref):
    emb = emb_ref[...]
    tabe_ref[...] = jnp.exp(-(emb @ wr_ref[...]))
    tabzh_ref[:, :H] = emb @ wzx_ref[...] + bz_ref[...]
    tabzh_ref[:, H:] = emb @ whx_ref[...] + bh_ref[...]
    tabo_ref[...] = emb @ wox_ref[...] + bo_ref[...]


def _make_tables(embedding, W_r, Wz_x, Wh_x, Wo_x, b_z, b_h, b_out):
    V = embedding.shape[0]
    return pl.pallas_call(
        _tables_body,
        out_shape=(jax.ShapeDtypeStruct((V, H), jnp.float32),
                   jax.ShapeDtypeStruct((V, 2 * H), jnp.float32),
                   jax.ShapeDtypeStruct((V, H), jnp.float32)),
    )(embedding, W_r.T, Wz_x.T, Wh_x.T, Wo_x.T,
      b_z[None, :], b_h[None, :], b_out[None, :])


def _gru_h(xzh_ref, sums_ref, wzh_ref, whh_ref, be):
    sumh = sums_ref[:, :H]
    z = jax.nn.sigmoid(xzh_ref[:, :H] + sumh @ wzh_ref[...])
    p = jnp.tanh(xzh_ref[:, H:] + sums_ref[:, H:] @ whh_ref[...])
    h = (1.0 - z) * sumh + z * p
    row = pl.program_id(0) * be + lax.broadcasted_iota(jnp.int32, (be, 1), 0)
    return jnp.where(row > 0, h, 0.0)


def _gru_body(xzh_ref, sums_ref, wzh_ref, whh_ref, ur_ref,
              bur_ref, hhu_ref, *, be):
    h = _gru_h(xzh_ref, sums_ref, wzh_ref, whh_ref, be)
    hhu_ref[:, :H] = h
    hhu_ref[:, H:] = jnp.exp(-(h @ ur_ref[...] + bur_ref[...]))


def _gru_last_body(xzh_ref, sums_ref, wzh_ref, whh_ref, h_ref, *, be):
    h_ref[...] = _gru_h(xzh_ref, sums_ref, wzh_ref, whh_ref, be)


def _gru_dense(xzh, sums, wzh, whh, ur_t, bur, *, be=4000):
    E = sums.shape[0]
    bs_e2 = pl.BlockSpec((be, 2 * H), lambda i: (i, 0))
    bs_w = pl.BlockSpec((H, H), lambda i: (0, 0))
    return pl.pallas_call(
        functools.partial(_gru_body, be=be),
        grid=(E // be,),
        in_specs=[bs_e2, bs_e2, bs_w, bs_w, bs_w,
                  pl.BlockSpec((1, H), lambda i: (0, 0))],
        out_specs=bs_e2,
        out_shape=jax.ShapeDtypeStruct((E, 2 * H), jnp.float32),
    )(xzh, sums, wzh, whh, ur_t, bur[None, :])


def _gru_dense_last(xzh, sums, wzh, whh, *, be=4000):
    E = sums.shape[0]
    bs_e2 = pl.BlockSpec((be, 2 * H), lambda i: (i, 0))
    bs_w = pl.BlockSpec((H, H), lambda i: (0, 0))
    return pl.pallas_call(
        functools.partial(_gru_last_body, be=be),
        grid=(E // be,),
        in_specs=[bs_e2, bs_e2, bs_w, bs_w],
        out_specs=pl.BlockSpec((be, H), lambda i: (i, 0)),
        out_shape=jax.ShapeDtypeStruct((E, H), jnp.float32),
    )(xzh, sums, wzh, whh)


def _root_body(eo_ref, msum_ref, woh_ref, tv_ref):
    tv_ref[...] = jnp.maximum(eo_ref[...] + msum_ref[...] @ woh_ref[...], 0.0)


def _root_out(eo, msum, woh_t):
    B = msum.shape[0]
    return pl.pallas_call(
        _root_body,
        out_shape=jax.ShapeDtypeStruct((B, H), jnp.float32),
    )(eo, msum, woh_t)


def kernel(node_wid_list, edge_node_idx_list, node_message_graph,
           mess_adjacency_graph, scope, embedding, W_z, b_z, W_r, U_r, b_Ur,
           W_h, b_h, W_out, b_out):
    E = mess_adjacency_graph.shape[0]
    N = node_wid_list.shape[0]

    Wz_x, Wz_h = W_z[:, :H], W_z[:, H:]
    Wh_x, Wh_h = W_h[:, :H], W_h[:, H:]
    Wo_x, Wo_h = W_out[:, :H], W_out[:, H:]

    tabe, tabzh, tabo = _make_tables(embedding, W_r, Wz_x, Wh_x, Wo_x,
                                     b_z, b_h, b_out)

    widx = jnp.take(node_wid_list, edge_node_idx_list, axis=0)
    exr, xzh = _sc_invariants(widx, tabe, tabzh)

    hhu = jnp.concatenate(
        [jnp.zeros((E, H), jnp.float32),
         jnp.broadcast_to(jnp.exp(-b_Ur)[None, :], (E, H))], axis=1)

    wzh_t, whh_t, ur_t, woh_t = Wz_h.T, Wh_h.T, U_r.T, Wo_h.T
    adjf = mess_adjacency_graph.reshape(E * NB)

    for it in range(DEPTH):
        sums = _sc_neighbor(hhu, exr, adjf)
        if it < DEPTH - 1:
            hhu = _gru_dense(xzh, sums, wzh_t, whh_t, ur_t, b_Ur)
        else:
            messages = _gru_dense_last(xzh, sums, wzh_t, whh_t)

    scope0 = scope[:, 0]
    swid = jnp.take(node_wid_list, scope0, axis=0)
    snmgf = jnp.take(node_message_graph, scope0, axis=0).reshape(-1)
    msum, eo = _sc_root(snmgf, swid, messages, tabo)
    tree_vecs = _root_out(eo, msum, woh_t)
    return (tree_vecs, messages)


# speedup vs baseline: 1.1370x; 1.1203x over previous
import functools

import jax
import jax.numpy as jnp
from jax import lax
from jax.experimental import pallas as pl
from jax.experimental.pallas import tpu as pltpu
from jax.experimental.pallas import tpu_sc as plsc

H = 128
NB = 4
DEPTH = 6

_NC, _NS, _L = 2, 16, 16
_NW = _NC * _NS


def _sc_invar_body(widx_hbm, tabe_hbm, tabzh_hbm, exr_hbm, xzh_hbm,
                   widx0, widx1, erows0, erows1, zrows0, zrows1,
                   semA0, semA1, semS0, semS1, *, ew, be):
    wid = lax.axis_index("s") * _NC + lax.axis_index("c")
    w0 = wid * ew
    nblk = ew // be
    tail = ew - nblk * be

    def fire(widx_v, erows_v, zrows_v, semA, base, cnt):
        pltpu.sync_copy(widx_hbm.at[pl.ds(base, cnt)],
                        widx_v.at[pl.ds(0, cnt)])
        pltpu.async_copy(tabe_hbm.at[widx_v.at[pl.ds(0, cnt)]],
                         erows_v.at[pl.ds(0, cnt)], semA)
        pltpu.async_copy(tabzh_hbm.at[widx_v.at[pl.ds(0, cnt)]],
                         zrows_v.at[pl.ds(0, cnt)], semA)

    def store(widx_v, erows_v, zrows_v, semA, semS, base, cnt):
        pltpu.make_async_copy(tabe_hbm.at[widx_v.at[pl.ds(0, cnt)]],
                              erows_v.at[pl.ds(0, cnt)], semA).wait()
        pltpu.make_async_copy(tabzh_hbm.at[widx_v.at[pl.ds(0, cnt)]],
                              zrows_v.at[pl.ds(0, cnt)], semA).wait()
        pltpu.async_copy(erows_v.at[pl.ds(0, cnt)],
                         exr_hbm.at[pl.ds(base, cnt)], semS)
        pltpu.async_copy(zrows_v.at[pl.ds(0, cnt)],
                         xzh_hbm.at[pl.ds(base, cnt)], semS)

    def wait_store(erows_v, zrows_v, semS, cnt):
        pltpu.make_async_copy(erows_v.at[pl.ds(0, cnt)],
                              exr_hbm.at[pl.ds(0, cnt)], semS).wait()
        pltpu.make_async_copy(zrows_v.at[pl.ds(0, cnt)],
                              xzh_hbm.at[pl.ds(0, cnt)], semS).wait()

    fire(widx0, erows0, zrows0, semA0, w0, be)
    fire(widx1, erows1, zrows1, semA1, w0 + be, be)
    store(widx0, erows0, zrows0, semA0, semS0, w0, be)

    def pair(i, c):
        b0 = w0 + (2 * i) * be
        b1 = b0 + be
        wait_store(erows0, zrows0, semS0, be)
        fire(widx0, erows0, zrows0, semA0, b0, be)
        store(widx1, erows1, zrows1, semA1, semS1, b1 - 2 * be, be)
        wait_store(erows1, zrows1, semS1, be)
        fire(widx1, erows1, zrows1, semA1, b1, be)
        store(widx0, erows0, zrows0, semA0, semS0, b0, be)
        return c

    lax.fori_loop(1, nblk // 2, pair, 0)
    store(widx1, erows1, zrows1, semA1, semS1, w0 + (nblk - 2) * be, be)
    wait_store(erows0, zrows0, semS0, be)
    fire(widx0, erows0, zrows0, semA0, w0 + (nblk - 1) * be, be)
    store(widx0, erows0, zrows0, semA0, semS0, w0 + (nblk - 1) * be, be)
    wait_store(erows1, zrows1, semS1, be)
    if tail:
        fire(widx1, erows1, zrows1, semA1, w0 + nblk * be, tail)
        store(widx1, erows1, zrows1, semA1, semS1, w0 + nblk * be, tail)
        wait_store(erows1, zrows1, semS1, tail)
    wait_store(erows0, zrows0, semS0, be)


def _sc_invariants(widx, tabe, tabzh, *, be=128):
    E = widx.shape[0]
    ew = E // _NW
    mesh = plsc.VectorSubcoreMesh(core_axis_name="c", subcore_axis_name="s")
    f = pl.kernel(
        functools.partial(_sc_invar_body, ew=ew, be=be),
        out_type=(jax.ShapeDtypeStruct((E, H), jnp.float32),
                  jax.ShapeDtypeStruct((E, 2 * H), jnp.float32)),
        mesh=mesh,
        scratch_types=[
            pltpu.VMEM((be,), jnp.int32),
            pltpu.VMEM((be,), jnp.int32),
            pltpu.VMEM((be, H), jnp.float32),
            pltpu.VMEM((be, H), jnp.float32),
            pltpu.VMEM((be, 2 * H), jnp.float32),
            pltpu.VMEM((be, 2 * H), jnp.float32),
            pltpu.SemaphoreType.DMA,
            pltpu.SemaphoreType.DMA,
            pltpu.SemaphoreType.DMA,
            pltpu.SemaphoreType.DMA,
        ],
    )
    return f(widx, tabe, tabzh)


def _sc_root_body(snmgf_hbm, swid_hbm, msgs_hbm, tabo_hbm, msum_hbm, eo_hbm,
                  idx_v, rows_v, widx_v, orows_v, msum_v, sem, *, bw):
    wid = lax.axis_index("s") * _NC + lax.axis_index("c")
    base = wid * bw
    pltpu.sync_copy(snmgf_hbm.at[pl.ds(base * NB, bw * NB)], idx_v)
    cp = pltpu.async_copy(msgs_hbm.at[idx_v], rows_v, sem)
    pltpu.sync_copy(swid_hbm.at[pl.ds(base, bw)], widx_v)
    cp.wait()
    pltpu.async_copy(tabo_hbm.at[widx_v], orows_v, sem).wait()
    for nloc in range(bw):
        r0 = nloc * NB
        for j in range(H // _L):
            o = j * _L
            s = rows_v[r0, pl.ds(o, _L)]
            for m in range(1, NB):
                s = s + rows_v[r0 + m, pl.ds(o, _L)]
            msum_v[nloc, pl.ds(o, _L)] = s
    pltpu.sync_copy(msum_v, msum_hbm.at[pl.ds(base, bw)])
    pltpu.sync_copy(orows_v, eo_hbm.at[pl.ds(base, bw)])


def _sc_root(snmgf, swid, msgs, tabo):
    B = swid.shape[0]
    bw = B // _NW
    mesh = plsc.VectorSubcoreMesh(core_axis_name="c", subcore_axis_name="s")
    f = pl.kernel(
        functools.partial(_sc_root_body, bw=bw),
        out_type=(jax.ShapeDtypeStruct((B, H), jnp.float32),
                  jax.ShapeDtypeStruct((B, H), jnp.float32)),
        mesh=mesh,
        scratch_types=[
            pltpu.VMEM((bw * NB,), jnp.int32),
            pltpu.VMEM((bw * NB, H), jnp.float32),
            pltpu.VMEM((bw,), jnp.int32),
            pltpu.VMEM((bw, H), jnp.float32),
            pltpu.VMEM((bw, H), jnp.float32),
            pltpu.SemaphoreType.DMA,
        ],
    )
    return f(snmgf, swid, msgs, tabo)


def _sc_nei_body(heu_hbm, exr_hbm, adjf_hbm, sums_hbm,
                 idx0, idx1, rows0, rows1, exr0, exr1, out0, out1,
                 semA0, semA1, semS0, semS1, *, ew, be):
    wid = lax.axis_index("s") * _NC + lax.axis_index("c")
    w0 = wid * ew
    nblk = ew // be

    def fire(idx_v, rows_v, exr_v, semA, base):
        pltpu.sync_copy(adjf_hbm.at[pl.ds(base * NB, be * NB)], idx_v)
        pltpu.async_copy(heu_hbm.at[idx_v], rows_v, semA)
        pltpu.async_copy(exr_hbm.at[pl.ds(base, be)], exr_v, semA)

    def wait_store(out_v, semS):
        pltpu.make_async_copy(out_v, sums_hbm.at[pl.ds(0, be)], semS).wait()

    def compute(idx_v, rows_v, exr_v, out_v, semA, semS, base):
        pltpu.make_async_copy(heu_hbm.at[idx_v], rows_v, semA).wait()
        pltpu.make_async_copy(exr_hbm.at[pl.ds(base, be)], exr_v, semA).wait()

        @plsc.parallel_loop(0, be, 1, unroll=2)
        def _edge(e):
            r0 = e * NB
            for j in range(H // _L):
                o = j * _L
                ex16 = exr_v[e, pl.ds(o, _L)]
                s = jnp.zeros((_L,), jnp.float32)
                g = jnp.zeros((_L,), jnp.float32)
                for m in range(NB):
                    hv = rows_v[r0 + m, pl.ds(o, _L)]
                    eu = rows_v[r0 + m, pl.ds(H + o, _L)]
                    s = s + hv
                    g = g + hv / (1.0 + ex16 * eu)
                out_v[e, pl.ds(o, _L)] = s
                out_v[e, pl.ds(H + o, _L)] = g

        pltpu.async_copy(out_v, sums_hbm.at[pl.ds(base, be)], semS)

    fire(idx0, rows0, exr0, semA0, w0)
    fire(idx1, rows1, exr1, semA1, w0 + be)
    compute(idx0, rows0, exr0, out0, semA0, semS0, w0)
    fire(idx0, rows0, exr0, semA0, w0 + 2 * be)
    compute(idx1, rows1, exr1, out1, semA1, semS1, w0 + be)

    def pair(i, carry):
        b0 = w0 + (2 * i) * be
        b1 = b0 + be
        fire(idx1, rows1, exr1, semA1, b1)
        wait_store(out0, semS0)
        compute(idx0, rows0, exr0, out0, semA0, semS0, b0)
        fire(idx0, rows0, exr0, semA0, b1 + be)
        wait_store(out1, semS1)
        compute(idx1, rows1, exr1, out1, semA1, semS1, b1)
        return carry

    lax.fori_loop(1, nblk // 2, pair, 0)
    wait_store(out0, semS0)
    compute(idx0, rows0, exr0, out0, semA0, semS0, w0 + (nblk - 1) * be)
    wait_store(out0, semS0)
    wait_store(out1, semS1)


def _sc_neighbor(heu, exr, adjf, *, be=40):
    E = exr.shape[0]
    ew = E // _NW
    mesh = plsc.VectorSubcoreMesh(core_axis_name="c", subcore_axis_name="s")
    f = pl.kernel(
        functools.partial(_sc_nei_body, ew=ew, be=be),
        out_type=jax.ShapeDtypeStruct((E, 2 * H), jnp.float32),
        mesh=mesh,
        scratch_types=[
            pltpu.VMEM((be * NB,), jnp.int32),
            pltpu.VMEM((be * NB,), jnp.int32),
            pltpu.VMEM((be * NB, 2 * H), jnp.float32),
            pltpu.VMEM((be * NB, 2 * H), jnp.float32),
            pltpu.VMEM((be, H), jnp.float32),
            pltpu.VMEM((be, H), jnp.float32),
            pltpu.VMEM((be, 2 * H), jnp.float32),
            pltpu.VMEM((be, 2 * H), jnp.float32),
            pltpu.SemaphoreType.DMA,
            pltpu.SemaphoreType.DMA,
            pltpu.SemaphoreType.DMA,
            pltpu.SemaphoreType.DMA,
        ],
    )
    return f(heu, exr, adjf)


def _tables_body(emb_ref, wr_ref, wzx_ref, whx_ref, wox_ref, bz_ref, bh_ref,
                 bo_ref, tabe_ref, tabzh_ref, tabo_ref):
    emb = emb_ref[...]
    tabe_ref[...] = jnp.exp(-(emb @ wr_ref[...]))
    tabzh_ref[:, :H] = emb @ wzx_ref[...] + bz_ref[...]
    tabzh_ref[:, H:] = emb @ whx_ref[...] + bh_ref[...]
    tabo_ref[...] = emb @ wox_ref[...] + bo_ref[...]


def _make_tables(embedding, W_r, Wz_x, Wh_x, Wo_x, b_z, b_h, b_out):
    V = embedding.shape[0]
    return pl.pallas_call(
        _tables_body,
        out_shape=(jax.ShapeDtypeStruct((V, H), jnp.float32),
                   jax.ShapeDtypeStruct((V, 2 * H), jnp.float32),
                   jax.ShapeDtypeStruct((V, H), jnp.float32)),
    )(embedding, W_r.T, Wz_x.T, Wh_x.T, Wo_x.T,
      b_z[None, :], b_h[None, :], b_out[None, :])


def _gru_h(xzh_ref, sums_ref, wzh_ref, whh_ref, be):
    sumh = sums_ref[:, :H]
    z = jax.nn.sigmoid(xzh_ref[:, :H] + sumh @ wzh_ref[...])
    p = jnp.tanh(xzh_ref[:, H:] + sums_ref[:, H:] @ whh_ref[...])
    h = (1.0 - z) * sumh + z * p
    row = pl.program_id(0) * be + lax.broadcasted_iota(jnp.int32, (be, 1), 0)
    return jnp.where(row > 0, h, 0.0)


def _gru_body(xzh_ref, sums_ref, wzh_ref, whh_ref, ur_ref,
              bur_ref, hhu_ref, *, be):
    h = _gru_h(xzh_ref, sums_ref, wzh_ref, whh_ref, be)
    hhu_ref[:, :H] = h
    hhu_ref[:, H:] = jnp.exp(-(h @ ur_ref[...] + bur_ref[...]))


def _gru_last_body(xzh_ref, sums_ref, wzh_ref, whh_ref, h_ref, *, be):
    h_ref[...] = _gru_h(xzh_ref, sums_ref, wzh_ref, whh_ref, be)


def _gru_first_body(xzh_ref, ur_ref, bur_ref, hhu_ref, *, be):
    h = jax.nn.sigmoid(xzh_ref[:, :H]) * jnp.tanh(xzh_ref[:, H:])
    row = pl.program_id(0) * be + lax.broadcasted_iota(jnp.int32, (be, 1), 0)
    h = jnp.where(row > 0, h, 0.0)
    hhu_ref[:, :H] = h
    hhu_ref[:, H:] = jnp.exp(-(h @ ur_ref[...] + bur_ref[...]))


def _gru_first(xzh, ur_t, bur, *, be=4000):
    E = xzh.shape[0]
    bs_e2 = pl.BlockSpec((be, 2 * H), lambda i: (i, 0))
    return pl.pallas_call(
        functools.partial(_gru_first_body, be=be),
        grid=(E // be,),
        in_specs=[bs_e2, pl.BlockSpec((H, H), lambda i: (0, 0)),
                  pl.BlockSpec((1, H), lambda i: (0, 0))],
        out_specs=bs_e2,
        out_shape=jax.ShapeDtypeStruct((E, 2 * H), jnp.float32),
    )(xzh, ur_t, bur[None, :])


def _gru_dense(xzh, sums, wzh, whh, ur_t, bur, *, be=4000):
    E = sums.shape[0]
    bs_e2 = pl.BlockSpec((be, 2 * H), lambda i: (i, 0))
    bs_w = pl.BlockSpec((H, H), lambda i: (0, 0))
    return pl.pallas_call(
        functools.partial(_gru_body, be=be),
        grid=(E // be,),
        in_specs=[bs_e2, bs_e2, bs_w, bs_w, bs_w,
                  pl.BlockSpec((1, H), lambda i: (0, 0))],
        out_specs=bs_e2,
        out_shape=jax.ShapeDtypeStruct((E, 2 * H), jnp.float32),
    )(xzh, sums, wzh, whh, ur_t, bur[None, :])


def _gru_dense_last(xzh, sums, wzh, whh, *, be=4000):
    E = sums.shape[0]
    bs_e2 = pl.BlockSpec((be, 2 * H), lambda i: (i, 0))
    bs_w = pl.BlockSpec((H, H), lambda i: (0, 0))
    return pl.pallas_call(
        functools.partial(_gru_last_body, be=be),
        grid=(E // be,),
        in_specs=[bs_e2, bs_e2, bs_w, bs_w],
        out_specs=pl.BlockSpec((be, H), lambda i: (i, 0)),
        out_shape=jax.ShapeDtypeStruct((E, H), jnp.float32),
    )(xzh, sums, wzh, whh)


def _root_body(eo_ref, msum_ref, woh_ref, tv_ref):
    tv_ref[...] = jnp.maximum(eo_ref[...] + msum_ref[...] @ woh_ref[...], 0.0)


def _root_out(eo, msum, woh_t):
    B = msum.shape[0]
    return pl.pallas_call(
        _root_body,
        out_shape=jax.ShapeDtypeStruct((B, H), jnp.float32),
    )(eo, msum, woh_t)


def kernel(node_wid_list, edge_node_idx_list, node_message_graph,
           mess_adjacency_graph, scope, embedding, W_z, b_z, W_r, U_r, b_Ur,
           W_h, b_h, W_out, b_out):
    E = mess_adjacency_graph.shape[0]
    N = node_wid_list.shape[0]

    Wz_x, Wz_h = W_z[:, :H], W_z[:, H:]
    Wh_x, Wh_h = W_h[:, :H], W_h[:, H:]
    Wo_x, Wo_h = W_out[:, :H], W_out[:, H:]

    tabe, tabzh, tabo = _make_tables(embedding, W_r, Wz_x, Wh_x, Wo_x,
                                     b_z, b_h, b_out)

    widx = jnp.take(node_wid_list, edge_node_idx_list, axis=0)
    exr, xzh = _sc_invariants(widx, tabe, tabzh)

    wzh_t, whh_t, ur_t, woh_t = Wz_h.T, Wh_h.T, U_r.T, Wo_h.T
    adjf = mess_adjacency_graph.reshape(E * NB)

    hhu = _gru_first(xzh, ur_t, b_Ur)

    for it in range(1, DEPTH):
        sums = _sc_neighbor(hhu, exr, adjf)
        if it < DEPTH - 1:
            hhu = _gru_dense(xzh, sums, wzh_t, whh_t, ur_t, b_Ur)
        else:
            messages = _gru_dense_last(xzh, sums, wzh_t, whh_t)

    scope0 = scope[:, 0]
    swid = jnp.take(node_wid_list, scope0, axis=0)
    snmgf = jnp.take(node_message_graph, scope0, axis=0).reshape(-1)
    msum, eo = _sc_root(snmgf, swid, messages, tabo)
    tree_vecs = _root_out(eo, msum, woh_t)
    return (tree_vecs, messages)
